# Initial kernel scaffold; baseline (speedup 1.0000x reference)
import jax
import jax.numpy as jnp
from jax.experimental import pallas as pl


def kernel(X, table, bias):
    raise NotImplementedError("write your pallas kernel here")


# speedup vs baseline: 1.4125x; 1.4125x over previous
import jax
import jax.numpy as jnp
from jax import lax
from jax.experimental import pallas as pl
from jax.experimental.pallas import tpu as pltpu
from jax.experimental.pallas import tpu_sc as plsc

_BATCH = 16384
_FIELDS = 26
_L = 16

_NW = 32
_ROWS_W = _BATCH // _NW
_IDX_W = _ROWS_W * _FIELDS
_CHUNK = 128
_NCHUNK = _IDX_W // _CHUNK
_NVREG = _ROWS_W // _L


def _body(xt, table, bias16, out, idx_v, vals_v, bias_v, acc_v, sem):
    wid = lax.axis_index("s") * 2 + lax.axis_index("c")

    pltpu.sync_copy(xt.at[wid], idx_v)
    pltpu.sync_copy(bias16, bias_v)

    def fire(j, _):
        pltpu.async_copy(
            table.at[idx_v.at[j]], vals_v.at[pl.ds(j * _CHUNK, _CHUNK)], sem
        )
        return None

    lax.fori_loop(0, _NCHUNK, fire, None)

    pltpu.make_async_copy(table.at[pl.ds(0, _IDX_W)], vals_v, sem).wait()

    bias_vec = bias_v[...]
    for i in range(_NVREG):
        acc = bias_vec
        for f in range(_FIELDS):
            acc = acc + vals_v[pl.ds(f * _ROWS_W + i * _L, _L)]
        acc_v[pl.ds(i * _L, _L)] = acc

    pltpu.sync_copy(acc_v, out.at[wid])


@jax.jit
def _lr_embed_sum(xt, table_flat, bias16):
    mesh = plsc.VectorSubcoreMesh(core_axis_name="c", subcore_axis_name="s")
    run = pl.kernel(
        _body,
        out_type=jax.ShapeDtypeStruct((_NW, _ROWS_W), jnp.float32),
        mesh=mesh,
        scratch_types=[
            pltpu.VMEM((_NCHUNK, _CHUNK), jnp.int32),
            pltpu.VMEM((_IDX_W,), jnp.float32),
            pltpu.VMEM((_L,), jnp.float32),
            pltpu.VMEM((_ROWS_W,), jnp.float32),
            pltpu.SemaphoreType.DMA,
        ],
    )
    return run(xt, table_flat, bias16)


def kernel(X, table, bias):
    xt = (
        X.astype(jnp.int32)
        .T.reshape(_FIELDS, _NW, 4, _CHUNK)
        .transpose(1, 0, 2, 3)
        .reshape(_NW, _NCHUNK, _CHUNK)
    )
    table_flat = table.reshape(-1)
    bias16 = jnp.broadcast_to(bias.astype(jnp.float32), (_L,))
    out = _lr_embed_sum(xt, table_flat, bias16)
    return out.reshape(_BATCH, 1)


# speedup vs baseline: 2.5945x; 1.8368x over previous
import jax
import jax.numpy as jnp
from jax import lax
from jax.experimental import pallas as pl
from jax.experimental.pallas import tpu as pltpu
from jax.experimental.pallas import tpu_sc as plsc

_BATCH = 16384
_FIELDS = 26
_L = 16

_NW = 32
_ROWS_W = _BATCH // _NW
_IDX_W = _ROWS_W * _FIELDS
_CHUNK = 128
_NCHUNK = _IDX_W // _CHUNK
_NVREG = _ROWS_W // _L

_VOCAB = 1000000
_VOCAB_PAD = 1024000


def _body(xt, table, bias16, out, idx_v, vals_v, bias_v, acc_v, sem):
    wid = lax.axis_index("s") * 2 + lax.axis_index("c")
    base = wid * _ROWS_W

    pltpu.sync_copy(xt.at[:, pl.ds(base, _ROWS_W)], idx_v)
    pltpu.sync_copy(bias16, bias_v)

    def fire(j, _):
        f = j // 4
        c = j - f * 4
        pltpu.async_copy(
            table.at[idx_v.at[f, pl.ds(c * _CHUNK, _CHUNK)]],
            vals_v.at[pl.ds(j * _CHUNK, _CHUNK)],
            sem,
        )
        return None

    lax.fori_loop(0, _NCHUNK, fire, None)

    pltpu.make_async_copy(table.at[pl.ds(0, _IDX_W)], vals_v, sem).wait()

    bias_vec = bias_v[...]
    for i in range(_NVREG):
        acc = bias_vec
        for f in range(_FIELDS):
            acc = acc + vals_v[pl.ds(f * _ROWS_W + i * _L, _L)]
        acc_v[pl.ds(i * _L, _L)] = acc

    pltpu.sync_copy(acc_v, out.at[pl.ds(base, _ROWS_W)])


@jax.jit
def _lr_embed_sum(xt, table_flat, bias16):
    mesh = plsc.VectorSubcoreMesh(core_axis_name="c", subcore_axis_name="s")
    run = pl.kernel(
        _body,
        out_type=jax.ShapeDtypeStruct((_BATCH,), jnp.float32),
        mesh=mesh,
        scratch_types=[
            pltpu.VMEM((_FIELDS, _ROWS_W), jnp.int32),
            pltpu.VMEM((_IDX_W,), jnp.float32),
            pltpu.VMEM((_L,), jnp.float32),
            pltpu.VMEM((_ROWS_W,), jnp.float32),
            pltpu.SemaphoreType.DMA,
        ],
    )
    return run(xt, table_flat, bias16)


def kernel(X, table, bias):
    xt = X.astype(jnp.int32).T
    table_flat = jnp.pad(table.T, ((0, 0), (0, _VOCAB_PAD - _VOCAB))).reshape(
        _VOCAB_PAD
    )
    bias16 = jnp.broadcast_to(bias.astype(jnp.float32), (_L,))
    out = _lr_embed_sum(xt, table_flat, bias16)
    return out.reshape(_BATCH, 1)


# speedup vs baseline: 2.6673x; 1.0281x over previous
import jax
import jax.numpy as jnp
from jax import lax
from jax.experimental import pallas as pl
from jax.experimental.pallas import tpu as pltpu
from jax.experimental.pallas import tpu_sc as plsc

_BATCH = 16384
_FIELDS = 26
_L = 16

_NW = 32
_ROWS_W = _BATCH // _NW
_IDX_W = _ROWS_W * _FIELDS
_CHUNK = 128
_NCHUNK = _IDX_W // _CHUNK
_NVREG = _ROWS_W // _L

_VOCAB = 1000000
_VOCAB_PAD = 1024000


def _body(xt, table, bias16, out, idx_v, vals_v, bias_v, acc_v, sem):
    wid = lax.axis_index("s") * 2 + lax.axis_index("c")
    base = wid * _ROWS_W

    pltpu.sync_copy(xt.at[:, pl.ds(base, _ROWS_W)], idx_v)
    pltpu.sync_copy(bias16, bias_v)

    def fire(j, _):
        f = j // 4
        c = j - f * 4
        pltpu.async_copy(
            table.at[idx_v.at[f, pl.ds(c * _CHUNK, _CHUNK)]],
            vals_v.at[pl.ds(j * _CHUNK, _CHUNK)],
            sem.at[f],
        )
        return None

    lax.fori_loop(0, _NCHUNK, fire, None)

    bias_vec = bias_v[...]
    for f in range(_FIELDS):
        pltpu.make_async_copy(
            out.at[pl.ds(0, _ROWS_W)],
            vals_v.at[pl.ds(f * _ROWS_W, _ROWS_W)],
            sem.at[f],
        ).wait()
        for i in range(_NVREG):
            v = vals_v[pl.ds(f * _ROWS_W + i * _L, _L)]
            if f == 0:
                acc_v[pl.ds(i * _L, _L)] = bias_vec + v
            else:
                acc_v[pl.ds(i * _L, _L)] = acc_v[pl.ds(i * _L, _L)] + v

    pltpu.sync_copy(acc_v, out.at[pl.ds(base, _ROWS_W)])


@jax.jit
def _lr_embed_sum(xt, table_flat, bias16):
    mesh = plsc.VectorSubcoreMesh(core_axis_name="c", subcore_axis_name="s")
    run = pl.kernel(
        _body,
        out_type=jax.ShapeDtypeStruct((_BATCH,), jnp.float32),
        mesh=mesh,
        scratch_types=[
            pltpu.VMEM((_FIELDS, _ROWS_W), jnp.int32),
            pltpu.VMEM((_IDX_W,), jnp.float32),
            pltpu.VMEM((_L,), jnp.float32),
            pltpu.VMEM((_ROWS_W,), jnp.float32),
            pltpu.SemaphoreType.DMA((_FIELDS,)),
        ],
    )
    return run(xt, table_flat, bias16)


def kernel(X, table, bias):
    xt = X.astype(jnp.int32).T
    table_flat = jnp.pad(table.T, ((0, 0), (0, _VOCAB_PAD - _VOCAB))).reshape(
        _VOCAB_PAD
    )
    bias16 = jnp.broadcast_to(bias.astype(jnp.float32), (_L,))
    out = _lr_embed_sum(xt, table_flat, bias16)
    return out.reshape(_BATCH, 1)


# speedup vs baseline: 2.7018x; 1.0129x over previous
import jax
import jax.numpy as jnp
from jax import lax
from jax.experimental import pallas as pl
from jax.experimental.pallas import tpu as pltpu
from jax.experimental.pallas import tpu_sc as plsc

_BATCH = 16384
_FIELDS = 26
_L = 16

_NW = 32
_ROWS_W = _BATCH // _NW
_IDX_W = _ROWS_W * _FIELDS
_CHUNK = 128
_NCHUNK = _IDX_W // _CHUNK
_NVREG = _ROWS_W // _L

_VOCAB = 1000000
_VOCAB_PAD = 1024000


def _body(xt, table, bias16, out, idx_v, vals_v, bias_v, acc_v, sem):
    wid = lax.axis_index("s") * 2 + lax.axis_index("c")
    base = wid * _ROWS_W

    pltpu.sync_copy(xt.at[:, pl.ds(base, _ROWS_W)], idx_v)
    pltpu.sync_copy(bias16, bias_v)

    def fire(j, _):
        f = j // 4
        c = j - f * 4
        pltpu.async_copy(
            table.at[idx_v.at[f, pl.ds(c * _CHUNK, _CHUNK)]],
            vals_v.at[pl.ds(j * _CHUNK, _CHUNK)],
            sem.at[f],
        )
        return None

    lax.fori_loop(0, _NCHUNK, fire, None)

    bias_vec = bias_v[...]

    def init_vregs(i, _):
        acc_v[pl.ds(i * _L, _L)] = bias_vec
        return None

    lax.fori_loop(0, _NVREG, init_vregs, None)

    def accum_field(f, _):
        pltpu.make_async_copy(
            out.at[pl.ds(0, _ROWS_W)],
            vals_v.at[pl.ds(f * _ROWS_W, _ROWS_W)],
            sem.at[f],
        ).wait()

        def accum_vreg(i, _):
            o = i * _L
            acc_v[pl.ds(o, _L)] = (
                acc_v[pl.ds(o, _L)] + vals_v[pl.ds(f * _ROWS_W + o, _L)]
            )
            return None

        lax.fori_loop(0, _NVREG, accum_vreg, None)
        return None

    lax.fori_loop(0, _FIELDS, accum_field, None)

    pltpu.sync_copy(acc_v, out.at[pl.ds(base, _ROWS_W)])


@jax.jit
def _lr_embed_sum(xt, table_flat, bias16):
    mesh = plsc.VectorSubcoreMesh(core_axis_name="c", subcore_axis_name="s")
    run = pl.kernel(
        _body,
        out_type=jax.ShapeDtypeStruct((_BATCH,), jnp.float32),
        mesh=mesh,
        scratch_types=[
            pltpu.VMEM((_FIELDS, _ROWS_W), jnp.int32),
            pltpu.VMEM((_IDX_W,), jnp.float32),
            pltpu.VMEM((_L,), jnp.float32),
            pltpu.VMEM((_ROWS_W,), jnp.float32),
            pltpu.SemaphoreType.DMA((_FIELDS,)),
        ],
    )
    return run(xt, table_flat, bias16)


def kernel(X, table, bias):
    xt = X.astype(jnp.int32).T
    table_flat = jnp.pad(table.T, ((0, 0), (0, _VOCAB_PAD - _VOCAB))).reshape(
        _VOCAB_PAD
    )
    bias16 = jnp.broadcast_to(bias.astype(jnp.float32), (_L,))
    out = _lr_embed_sum(xt, table_flat, bias16)
    return out.reshape(_BATCH, 1)


# speedup vs baseline: 2.7104x; 1.0032x over previous
import jax
import jax.numpy as jnp
from jax import lax
from jax.experimental import pallas as pl
from jax.experimental.pallas import tpu as pltpu
from jax.experimental.pallas import tpu_sc as plsc

_BATCH = 16384
_FIELDS = 26
_L = 16

_NW = 32
_ROWS_W = _BATCH // _NW
_IDX_W = _ROWS_W * _FIELDS
_CHUNK = 128
_NCHUNK = _IDX_W // _CHUNK
_NVREG = _ROWS_W // _L

_VOCAB = 1000000
_VOCAB_PAD = 1024000


def _body(xt, table, bias16, out, idx_v, vals_v, bias_v, acc_v, sem):
    wid = lax.axis_index("s") * 2 + lax.axis_index("c")
    base = wid * _ROWS_W

    pltpu.sync_copy(xt.at[:, pl.ds(base, _ROWS_W)], idx_v)
    pltpu.sync_copy(bias16, bias_v)

    def fire(f, _):
        for c in range(4):
            pltpu.async_copy(
                table.at[idx_v.at[f, pl.ds(c * _CHUNK, _CHUNK)]],
                vals_v.at[pl.ds(f * _ROWS_W + c * _CHUNK, _CHUNK)],
                sem.at[f],
            )
        return None

    lax.fori_loop(0, _FIELDS, fire, None)

    bias_vec = bias_v[...]

    def init_vregs(i, _):
        acc_v[pl.ds(i * _L, _L)] = bias_vec
        return None

    lax.fori_loop(0, _NVREG, init_vregs, None)

    def accum_field(f, _):
        pltpu.make_async_copy(
            out.at[pl.ds(0, _ROWS_W)],
            vals_v.at[pl.ds(f * _ROWS_W, _ROWS_W)],
            sem.at[f],
        ).wait()

        def accum_vreg(i, _):
            o = i * _L
            acc_v[pl.ds(o, _L)] = (
                acc_v[pl.ds(o, _L)] + vals_v[pl.ds(f * _ROWS_W + o, _L)]
            )
            return None

        lax.fori_loop(0, _NVREG, accum_vreg, None)
        return None

    lax.fori_loop(0, _FIELDS, accum_field, None)

    pltpu.sync_copy(acc_v, out.at[pl.ds(base, _ROWS_W)])


@jax.jit
def _lr_embed_sum(xt, table_flat, bias16):
    mesh = plsc.VectorSubcoreMesh(core_axis_name="c", subcore_axis_name="s")
    run = pl.kernel(
        _body,
        out_type=jax.ShapeDtypeStruct((_BATCH,), jnp.float32),
        mesh=mesh,
        scratch_types=[
            pltpu.VMEM((_FIELDS, _ROWS_W), jnp.int32),
            pltpu.VMEM((_IDX_W,), jnp.float32),
            pltpu.VMEM((_L,), jnp.float32),
            pltpu.VMEM((_ROWS_W,), jnp.float32),
            pltpu.SemaphoreType.DMA((_FIELDS,)),
        ],
    )
    return run(xt, table_flat, bias16)


def kernel(X, table, bias):
    xt = X.astype(jnp.int32).T
    table_flat = jnp.pad(table.T, ((0, 0), (0, _VOCAB_PAD - _VOCAB))).reshape(
        _VOCAB_PAD
    )
    bias16 = jnp.broadcast_to(bias.astype(jnp.float32), (_L,))
    out = _lr_embed_sum(xt, table_flat, bias16)
    return out.reshape(_BATCH, 1)
